# baseline (speedup 1.0000x reference)
import functools

import jax
import jax.numpy as jnp
from jax import lax
from jax.experimental import pallas as pl
from jax.experimental.pallas import tpu as pltpu
from jax.experimental.pallas import tpu_sc as plsc

HALF = 192
NEMB = 32
NW = 32
CHUNK = 128
NLANE = 16


def _build_sc_call(n_pad, cpw, n_chunks):
    mesh = plsc.VectorSubcoreMesh(core_axis_name="c", subcore_axis_name="s")

    @functools.partial(
        pl.kernel,
        mesh=mesh,
        out_type=jax.ShapeDtypeStruct((n_pad, 2 * HALF), jnp.float32),
        scratch_types=[
            pltpu.VMEM((NEMB * HALF,), jnp.float32),
            pltpu.VMEM((NEMB * HALF,), jnp.float32),
            pltpu.VMEM((cpw,), jnp.int32),
            pltpu.VMEM((cpw,), jnp.int32),
            pltpu.VMEM((2, CHUNK, 2 * HALF), jnp.float32),
            pltpu.SemaphoreType.DMA,
        ],
    )
    def sc_fn(xt_hbm, yt_hbm, xd_hbm, yd_hbm, out_hbm, xt_v, yt_v, xd_v,
              yd_v, rows, sem_w):
        wid = lax.axis_index("s") * 2 + lax.axis_index("c")
        pltpu.sync_copy(xt_hbm, xt_v)
        pltpu.sync_copy(yt_hbm, yt_v)
        pltpu.sync_copy(xd_hbm.at[wid], xd_v)
        pltpu.sync_copy(yd_hbm.at[wid], yd_v)

        def write(c, buf):
            rowbase = wid * cpw + c * CHUNK
            return pltpu.make_async_copy(rows.at[buf],
                                         out_hbm.at[pl.ds(rowbase, CHUNK)],
                                         sem_w)

        def assemble(c, buf):
            def group_body(g, carry):
                xv = xd_v[pl.ds(c * CHUNK + g * NLANE, NLANE)] * HALF
                yv = yd_v[pl.ds(c * CHUNK + g * NLANE, NLANE)] * HALF
                for i in range(NLANE):
                    xoff = xv[i]
                    yoff = yv[i]
                    for k in range(HALF // NLANE):
                        rows[buf, g * NLANE + i, pl.ds(k * NLANE, NLANE)] = (
                            xt_v[pl.ds(xoff + k * NLANE, NLANE)])
                        rows[buf, g * NLANE + i,
                             pl.ds(HALF + k * NLANE, NLANE)] = (
                                 yt_v[pl.ds(yoff + k * NLANE, NLANE)])
                return carry

            lax.fori_loop(0, CHUNK // NLANE, group_body, 0)

        def body(c, carry):
            @pl.when(c >= 2)
            def _():
                write(c - 2, c % 2).wait()

            assemble(c, c % 2)
            write(c, c % 2).start()
            return carry

        lax.fori_loop(0, n_chunks, body, 0)
        write(n_chunks - 2, n_chunks % 2).wait()
        write(n_chunks - 1, (n_chunks - 1) % 2).wait()

    return sc_fn


def kernel(x_table, y_table, x_distances, y_distances):
    s = x_distances.shape[0]
    n = s * s
    n_chunks = -(-n // (CHUNK * NW))
    cpw = n_chunks * CHUNK
    n_pad = cpw * NW

    xd = jnp.zeros((n_pad,), jnp.int32).at[:n].set(
        x_distances.reshape(-1)).reshape(NW, cpw)
    yd = jnp.zeros((n_pad,), jnp.int32).at[:n].set(
        y_distances.reshape(-1)).reshape(NW, cpw)

    out = _build_sc_call(n_pad, cpw, n_chunks)(
        x_table.reshape(-1), y_table.reshape(-1), xd, yd)
    return out[:n].reshape(s, s, 2 * HALF)
